# initial kernel scaffold (unmeasured)
import jax
import jax.numpy as jnp
from jax import lax
from jax.experimental import pallas as pl
from jax.experimental.pallas import tpu as pltpu

N_DEV = 16


def kernel(x, w_mat):
    m_per, k = x.shape
    n = w_mat.shape[1]
    n_per = n // N_DEV
    m = m_per * N_DEV

    def body(x_ref, w_ref, out_ref, y_ref, amax_ref, comm_ref,
             dsend, drecv, asend, arecv):
        my = lax.axis_index("i")

        y = jnp.dot(x_ref[:, :], w_ref[:, :],
                    preferred_element_type=jnp.float32)
        y_ref[:, :] = y
        local_amax = jnp.max(jnp.abs(y))
        amax_ref[0:1, :] = jnp.full((1, 128), local_amax, jnp.float32)

        data_rdmas = []
        amax_rdmas = []
        for off in range(1, N_DEV):
            j = (my + off) % N_DEV
            r = pltpu.make_async_remote_copy(
                src_ref=y_ref.at[:, pl.ds(j * n_per, n_per)],
                dst_ref=comm_ref.at[off],
                send_sem=dsend.at[off],
                recv_sem=drecv.at[off],
                device_id=(j,),
                device_id_type=pl.DeviceIdType.MESH,
            )
            r.start()
            data_rdmas.append(r)
            a = pltpu.make_async_remote_copy(
                src_ref=amax_ref.at[pl.ds(0, 1)],
                dst_ref=amax_ref.at[pl.ds(off, 1)],
                send_sem=asend.at[off],
                recv_sem=arecv.at[off],
                device_id=(j,),
                device_id_type=pl.DeviceIdType.MESH,
            )
            a.start()
            amax_rdmas.append(a)

        out_ref[pl.ds(my * m_per, m_per), :] = y_ref[:, pl.ds(my * n_per, n_per)]

        for idx, off in enumerate(range(1, N_DEV)):
            data_rdmas[idx].wait()
            src_pos = (my - off) % N_DEV
            out_ref[pl.ds(src_pos * m_per, m_per), :] = comm_ref[off]
        for a in amax_rdmas:
            a.wait()

        g_amax = jnp.max(amax_ref[:, :])
        scale = g_amax / 448.0
        yq = jnp.clip(out_ref[:, :] / scale, -448.0, 448.0)
        yq = yq.astype(jnp.float8_e4m3fn).astype(jnp.float32)
        out_ref[:, :] = yq * scale

    return pl.pallas_call(
        body,
        out_shape=jax.ShapeDtypeStruct((m, n_per), jnp.float32),
        in_specs=[
            pl.BlockSpec(memory_space=pltpu.VMEM),
            pl.BlockSpec(memory_space=pltpu.VMEM),
        ],
        out_specs=pl.BlockSpec(memory_space=pltpu.VMEM),
        scratch_shapes=[
            pltpu.VMEM((m_per, n), jnp.float32),
            pltpu.VMEM((N_DEV, 128), jnp.float32),
            pltpu.VMEM((N_DEV, m_per, n_per), jnp.float32),
            pltpu.SemaphoreType.DMA((N_DEV,)),
            pltpu.SemaphoreType.DMA((N_DEV,)),
            pltpu.SemaphoreType.DMA((N_DEV,)),
            pltpu.SemaphoreType.DMA((N_DEV,)),
        ],
        compiler_params=pltpu.CompilerParams(collective_id=0),
    )(x, w_mat)


# baseline (device time: 51822 ns/iter reference)
import jax
import jax.numpy as jnp
from jax import lax
from jax.experimental import pallas as pl
from jax.experimental.pallas import tpu as pltpu

N_DEV = 16


def kernel(x, w_mat):
    m_per, k = x.shape
    n = w_mat.shape[1]
    n_per = n // N_DEV
    m = m_per * N_DEV

    def body(x_ref, w_ref, out_ref, y_ref, amax_ref, comm_ref,
             dsend, drecv, asend, arecv):
        my = lax.axis_index("i")

        y = jnp.dot(x_ref[:, :], w_ref[:, :],
                    preferred_element_type=jnp.float32)
        y_ref[:, :] = y
        local_amax = jnp.max(jnp.abs(y))
        amax_ref[0:1, :] = jnp.full((1, 128), local_amax, jnp.float32)

        data_rdmas = []
        amax_rdmas = []
        for off in range(1, N_DEV):
            j = (my + off) % N_DEV
            r = pltpu.make_async_remote_copy(
                src_ref=y_ref.at[:, pl.ds(j * n_per, n_per)],
                dst_ref=comm_ref.at[off],
                send_sem=dsend.at[off],
                recv_sem=drecv.at[off],
                device_id=(j,),
                device_id_type=pl.DeviceIdType.MESH,
            )
            r.start()
            data_rdmas.append(r)
            a = pltpu.make_async_remote_copy(
                src_ref=amax_ref.at[pl.ds(0, 1)],
                dst_ref=amax_ref.at[pl.ds(off, 1)],
                send_sem=asend.at[off],
                recv_sem=arecv.at[off],
                device_id=(j,),
                device_id_type=pl.DeviceIdType.MESH,
            )
            a.start()
            amax_rdmas.append(a)

        out_ref[pl.ds(my * m_per, m_per), :] = y_ref[:, pl.ds(my * n_per, n_per)]

        for idx, off in enumerate(range(1, N_DEV)):
            data_rdmas[idx].wait()
            src_pos = (my - off) % N_DEV
            out_ref[pl.ds(src_pos * m_per, m_per), :] = comm_ref[off]
        for a in amax_rdmas:
            a.wait()

        g_amax = jnp.max(amax_ref[:, :])
        scale = g_amax / 448.0
        yq = jnp.clip(out_ref[:, :] / scale, -448.0, 448.0)
        yq = yq.astype(jnp.float8_e4m3fn).astype(jnp.float32)
        out_ref[:, :] = yq * scale

    return pl.pallas_call(
        body,
        out_shape=jax.ShapeDtypeStruct((m, n_per), jnp.float32),
        in_specs=[
            pl.BlockSpec(memory_space=pltpu.VMEM),
            pl.BlockSpec(memory_space=pltpu.VMEM),
        ],
        out_specs=pl.BlockSpec(memory_space=pltpu.VMEM),
        scratch_shapes=[
            pltpu.VMEM((m_per, n), jnp.float32),
            pltpu.VMEM((N_DEV, 128), jnp.float32),
            pltpu.VMEM((N_DEV, m_per, n_per), jnp.float32),
            pltpu.SemaphoreType.DMA((N_DEV,)),
            pltpu.SemaphoreType.DMA((N_DEV,)),
            pltpu.SemaphoreType.DMA((N_DEV,)),
            pltpu.SemaphoreType.DMA((N_DEV,)),
        ],
        compiler_params=pltpu.CompilerParams(
            vmem_limit_bytes=100 * 1024 * 1024,
        ),
    )(x, w_mat)
